# Initial kernel scaffold; baseline (speedup 1.0000x reference)
import jax
import jax.numpy as jnp
from jax.experimental import pallas as pl
from jax.experimental.pallas import tpu as pltpu

MEM = 8192
DIM = 4096
NCHUNK = 8


def _chunks():
    base = (MEM - 1) // NCHUNK
    rem = (MEM - 1) % NCHUNK
    out = []
    start = 0
    for i in range(NCHUNK):
        size = base + (1 if i < rem else 0)
        out.append((start, size))
        start += size
    return out


def _shift_kernel(inp_ref, buf_ref, out_ref, row_sem, sems):
    copies = []
    for i, (start, size) in enumerate(_chunks()):
        c = pltpu.make_async_copy(
            buf_ref.at[pl.ds(start + 1, size)],
            out_ref.at[pl.ds(start, size)],
            sems.at[i],
        )
        c.start()
        copies.append(c)
    row = pltpu.make_async_copy(inp_ref, out_ref.at[MEM - 1], row_sem)
    row.start()
    for c in copies:
        c.wait()
    row.wait()


def kernel(inputs, memory_buffer):
    return pl.pallas_call(
        _shift_kernel,
        out_shape=jax.ShapeDtypeStruct((MEM, DIM), jnp.float32),
        in_specs=[
            pl.BlockSpec(memory_space=pltpu.ANY),
            pl.BlockSpec(memory_space=pltpu.ANY),
        ],
        out_specs=pl.BlockSpec(memory_space=pltpu.ANY),
        scratch_shapes=[
            pltpu.SemaphoreType.DMA,
            pltpu.SemaphoreType.DMA((NCHUNK,)),
        ],
    )(inputs, memory_buffer)


# speedup vs baseline: 6.0679x; 6.0679x over previous
import jax
import jax.numpy as jnp
from jax.experimental import pallas as pl
from jax.experimental.pallas import tpu as pltpu

MEM = 8192
DIM = 4096
R = 512
N = MEM // R


def _shift_kernel(inp_ref, a_ref, b_ref, o_ref):
    i = pl.program_id(0)
    o_ref[0 : R - 1, :] = a_ref[1:R, :]
    o_ref[R - 1 : R, :] = jnp.where(i == N - 1, inp_ref[...], b_ref[0:1, :])


def kernel(inputs, memory_buffer):
    return pl.pallas_call(
        _shift_kernel,
        grid=(N,),
        out_shape=jax.ShapeDtypeStruct((MEM, DIM), jnp.float32),
        in_specs=[
            pl.BlockSpec((1, DIM), lambda i: (0, 0)),
            pl.BlockSpec((R, DIM), lambda i: (i, 0)),
            pl.BlockSpec(
                (8, DIM),
                lambda i: (jnp.minimum((i + 1) * (R // 8), MEM // 8 - 1), 0),
            ),
        ],
        out_specs=pl.BlockSpec((R, DIM), lambda i: (i, 0)),
    )(inputs.reshape(1, DIM), memory_buffer, memory_buffer)
